# Initial kernel scaffold; baseline (speedup 1.0000x reference)
import jax
import jax.numpy as jnp
from jax.experimental import pallas as pl


def kernel(x, W_embed, b_embed, W_update, b_update, gamma, beta, W_out, b_out, context_strength):
    raise NotImplementedError("write your pallas kernel here")


# speedup vs baseline: 47.9267x; 47.9267x over previous
import functools

import jax
import jax.numpy as jnp
import numpy as np
from jax.experimental import pallas as pl
from jax.experimental.pallas import tpu as pltpu

B, T = 4096, 50
D = 256
M = 64
K = 2
TEMP = 8.0

OUT_N = 10
OUT_PAD = 128
B_BLK = 1024


def _context_coeffs():
    offsets = np.arange(-K, K + 1)
    w = np.zeros((T, 2 * K + 1), np.float32)
    for tp in range(T):
        ptr = np.float32(tp)
        base = int(np.clip(np.floor(ptr), 0, M - 1))
        idx = (base + offsets) % M
        delta = idx.astype(np.float32) - ptr
        logits = (-(delta ** 2) / np.float32(TEMP)).astype(np.float32)
        e = np.exp((logits - logits.max()).astype(np.float32)).astype(np.float32)
        w[tp] = (e / e.sum()).astype(np.float32)
    c1 = np.zeros(T, np.float32)
    c2 = np.zeros(T, np.float32)
    for t in range(T):
        if t >= 1:
            c1[t] = w[t - 1][K + 1]
        if t >= 2:
            c2[t] = w[t - 2][K + 2]
    return c1, c2


_C1, _C2 = _context_coeffs()


def _body(x_ref, we_ref, be_ref, wu_ref, bu_ref, g_ref, b_ref,
          wo_ref, bo_ref, cs_ref, out_ref):
    s = jax.nn.sigmoid(cs_ref[0, 0])
    we = we_ref[...]
    be = be_ref[...]
    wu = wu_ref[...]
    bu = bu_ref[...]
    gam = g_ref[...]
    bet = b_ref[...]
    h1 = jnp.zeros((B_BLK, D), jnp.float32)
    h2 = jnp.zeros((B_BLK, D), jnp.float32)
    for t in range(T):
        xt = x_ref[:, t:t + 1]
        inp = jnp.tanh(xt * we + be)
        combined = inp + (s * _C1[t]) * h1 + (s * _C2[t]) * h2 + h1
        hm = jnp.tanh(
            jax.lax.dot_general(combined, wu, (((1,), (0,)), ((), ())),
                                preferred_element_type=jnp.float32) + bu)
        mu = jnp.mean(hm, axis=-1, keepdims=True)
        var = jnp.mean((hm - mu) ** 2, axis=-1, keepdims=True)
        h = (hm - mu) / jnp.sqrt(var + 1e-5) * gam + bet
        h2 = h1
        h1 = h
    out_ref[...] = jax.lax.dot_general(
        h1, wo_ref[...], (((1,), (0,)), ((), ())),
        preferred_element_type=jnp.float32) + bo_ref[...]


@jax.jit
def kernel(x, W_embed, b_embed, W_update, b_update, gamma, beta,
           W_out, b_out, context_strength):
    x2 = x.reshape(B, T)
    wo_p = jnp.zeros((D, OUT_PAD), jnp.float32).at[:, :OUT_N].set(W_out)
    bo_p = jnp.zeros((1, OUT_PAD), jnp.float32).at[0, :OUT_N].set(b_out)
    cs = context_strength.reshape(1, 1)

    full = lambda shape: pl.BlockSpec(shape, lambda i: (0, 0))
    out = pl.pallas_call(
        _body,
        grid=(B // B_BLK,),
        in_specs=[
            pl.BlockSpec((B_BLK, T), lambda i: (i, 0)),
            full((1, D)), full((1, D)), full((D, D)), full((1, D)),
            full((1, D)), full((1, D)), full((D, OUT_PAD)),
            full((1, OUT_PAD)), full((1, 1)),
        ],
        out_specs=pl.BlockSpec((B_BLK, OUT_PAD), lambda i: (i, 0)),
        out_shape=jax.ShapeDtypeStruct((B, OUT_PAD), jnp.float32),
        compiler_params=pltpu.CompilerParams(
            dimension_semantics=("arbitrary",)),
    )(x2, W_embed, b_embed.reshape(1, D), W_update, b_update.reshape(1, D),
      gamma.reshape(1, D), beta.reshape(1, D), wo_p, bo_p, cs)
    return out[:, :OUT_N]
